# baseline (speedup 1.0000x reference)
import functools

import jax
import jax.numpy as jnp
from jax import lax
from jax.experimental import pallas as pl
from jax.experimental.pallas import tpu as pltpu
from jax.experimental.pallas import tpu_sc as plsc

_N = 256
_D = 256
_TOPK = 5
_L = 16
_NB = _D // _L


def _splat_f(v):
    return jnp.full((_L,), v, jnp.float32)


def _splat_i(v):
    return jnp.full((_L,), v, jnp.int32)


def _tree_reduce(vals, op):
    vals = list(vals)
    while len(vals) > 1:
        nxt = [op(vals[i], vals[i + 1]) for i in range(0, len(vals) - 1, 2)]
        if len(vals) % 2:
            nxt.append(vals[-1])
        vals = nxt
    return vals[0]


def _sc_top5_body(feat_hbm, ka_hbm, kb_hbm, rows_v, ka_v, kb_v):
    cid = lax.axis_index("c")
    sid = lax.axis_index("s")

    @pl.when(cid == 0)
    def _():
        base = sid * _L
        pltpu.sync_copy(feat_hbm.at[pl.ds(base, _L)], rows_v)
        lane = lax.iota(jnp.int32, _L)
        neg = _splat_f(-jnp.inf)

        bmax = []
        for k in range(_NB):
            cols = [plsc.load_gather(rows_v, [lane, _splat_i(k * _L + c)])
                    for c in range(_L)]
            bmax.append(_tree_reduce(cols, jnp.maximum))

        picked = []
        for _ in range(_TOPK):
            m = _tree_reduce(bmax, jnp.maximum)
            bidx = _tree_reduce(
                [jnp.where(bmax[k] == m, _splat_i(k), _splat_i(_NB))
                 for k in range(_NB)], jnp.minimum)
            gcol = bidx * _L
            gs = [plsc.load_gather(rows_v, [lane, gcol + c])
                  for c in range(_L)]
            cidx = _tree_reduce(
                [jnp.where(gs[c] == m, _splat_i(c), _splat_i(_L))
                 for c in range(_L)], jnp.minimum)
            idx = gcol + cidx
            picked.append(idx)
            plsc.store_scatter(rows_v, [lane, idx], neg)
            nb = _tree_reduce(
                [jnp.where(cidx == c, neg, gs[c]) for c in range(_L)],
                jnp.maximum)
            bmax = [jnp.where(bidx == k, nb, bmax[k]) for k in range(_NB)]

        s = picked
        for a, b in ((0, 1), (3, 4), (2, 4), (2, 3), (0, 3),
                     (0, 2), (1, 4), (1, 3), (1, 2)):
            lo = jnp.minimum(s[a], s[b])
            hi = jnp.maximum(s[a], s[b])
            s[a], s[b] = lo, hi
        f = [v.astype(jnp.float32) for v in s]
        klo = f[0] * 65536.0 + f[1] * 256.0 + f[2]
        khi = f[3] * 256.0 + f[4]

        plsc.store_scatter(ka_v, [lane, _splat_i(0)], klo)
        plsc.store_scatter(ka_v, [lane, _splat_i(1)], khi)
        kb_v[0, :] = klo
        kb_v[1, :] = khi
        pltpu.sync_copy(ka_v, ka_hbm.at[pl.ds(base, _L)])
        pltpu.sync_copy(kb_v, kb_hbm.at[:, pl.ds(base, _L)])


@functools.cache
def _sc_top5():
    return pl.kernel(
        _sc_top5_body,
        out_type=(jax.ShapeDtypeStruct((_N, 2), jnp.float32),
                  jax.ShapeDtypeStruct((2, _N), jnp.float32)),
        mesh=plsc.VectorSubcoreMesh(core_axis_name="c",
                                    subcore_axis_name="s"),
        scratch_types=[
            pltpu.VMEM((_L, _D), jnp.float32),
            pltpu.VMEM((_L, 2), jnp.float32),
            pltpu.VMEM((2, _L), jnp.float32),
        ],
        compiler_params=pltpu.CompilerParams(use_tc_tiling_on_sc=False,
                                             needs_layout_passes=False),
    )


def _tc_loss_kernel(prob1_ref, prob2_ref, ka_ref, kb_ref, out_ref):
    klo_c = ka_ref[:, 0:1]
    khi_c = ka_ref[:, 1:2]
    klo_r = kb_ref[0:1, :]
    khi_r = kb_ref[1:2, :]
    target = (klo_c == klo_r) & (khi_c == khi_r)

    sim = jax.lax.dot_general(
        prob2_ref[...].astype(jnp.bfloat16),
        prob1_ref[...].astype(jnp.bfloat16),
        (((1,), (1,)), ((), ())),
        preferred_element_type=jnp.float32)
    eps = 1e-12
    p = jnp.clip(sim, eps, 1.0 - eps)
    q = jnp.where(target, p, 1.0 - p)
    out_ref[0, 0] = -jnp.sum(jnp.log(q)) / (_N * _N)


def kernel(feat1, feat2, prob1, prob2):
    del feat2
    ka, kb = _sc_top5()(feat1)
    out = pl.pallas_call(
        _tc_loss_kernel,
        out_shape=jax.ShapeDtypeStruct((1, 1), jnp.float32),
        out_specs=pl.BlockSpec(memory_space=pltpu.SMEM),
    )(prob1, prob2, ka, kb)
    return out.reshape(())
